# baseline (speedup 1.0000x reference)
import functools

import numpy as np
import jax
import jax.numpy as jnp
from jax.experimental import pallas as pl
from jax.experimental.pallas import tpu as pltpu

_S = 8
_BR = 8
_TV = 3072

_KS0 = np.uint32(0)
_KS1 = np.uint32(42)
_KS2 = np.uint32(int(_KS0) ^ int(_KS1) ^ 0x1BD11BDA)
_KSCHED = (_KS1, _KS2, _KS0, _KS1, _KS2, _KS0)
_ROT = ((13, 15, 26, 6), (17, 29, 16, 24))
_TINY = np.float32(np.finfo(np.float32).tiny)


def _threefry_bits(x1):
    x0 = jnp.zeros_like(x1)
    x1 = x1 + _KS1
    for i in range(5):
        for r in _ROT[i % 2]:
            x0 = x0 + x1
            x1 = ((x1 << np.uint32(r)) | (x1 >> np.uint32(32 - r))) ^ x0
        x0 = x0 + _KSCHED[i]
        x1 = x1 + np.uint32(int(_KSCHED[i + 1]) + i + 1)
    return x0 ^ x1


def _bits_to_exp(bits):
    fb = (bits >> np.uint32(9)) | np.uint32(0x3F800000)
    f = jax.lax.bitcast_convert_type(fb, jnp.float32) - np.float32(1.0)
    return -jnp.log(f + _TINY)


def _body(nt, n_b, v_size, p_ref, m_ref, act_ref, sd_ref,
          vmin_ref, vt_ref, rsum_ref):
    g = pl.program_id(0)
    t = pl.program_id(1)

    @pl.when(t == 0)
    def _init():
        vmin_ref[...] = jnp.full((_S, _BR, _TV), jnp.inf, jnp.float32)
        vt_ref[...] = jnp.zeros((_S, _BR, _TV), jnp.int32)
        rsum_ref[...] = jnp.zeros((_BR, _TV), jnp.float32)

    p = p_ref[...]
    m = m_ref[...]
    brow = jax.lax.broadcasted_iota(jnp.int32, (_BR, _TV), 0)
    lane = jax.lax.broadcasted_iota(jnp.int32, (_BR, _TV), 1)
    v = lane + t * np.int32(_TV)
    valid = v < np.int32(v_size)
    mf = jnp.where(v == 0, np.float32(1.0), m.astype(jnp.float32))
    pm = jnp.where(valid, (p + np.float32(1e-14)) * mf, np.float32(0.0))
    rsum_ref[...] += pm
    rpm = np.float32(1.0) / pm
    rowbase = (g * np.int32(_BR) + brow) * np.int32(v_size) + v
    for s in range(_S):
        flat = rowbase + np.int32(s * n_b * v_size)
        e = _bits_to_exp(_threefry_bits(flat.astype(jnp.uint32)))
        score = e * rpm
        old = vmin_ref[s]
        upd = score < old
        vt_ref[s] = jnp.where(upd, t, vt_ref[s])
        vmin_ref[s] = jnp.minimum(score, old)

    @pl.when(t == nt - 1)
    def _fin():
        ssum = jnp.sum(rsum_ref[...], axis=1)
        acts = []
        mvs = []
        for s in range(_S):
            vm = vmin_ref[s]
            mv = jnp.min(vm, axis=1)
            sel = vm == mv[:, None]
            cand = jnp.where(sel, vt_ref[s] * np.int32(_TV) + lane,
                             np.int32(2**31 - 1))
            acts.append(jnp.min(cand, axis=1))
            mvs.append(mv)
        act = jnp.stack(acts, axis=0)
        mvm = jnp.stack(mvs, axis=0)
        srow = jax.lax.broadcasted_iota(jnp.int32, (_S, _BR), 0)
        bcol = jax.lax.broadcasted_iota(jnp.int32, (_S, _BR), 1)
        flatw = ((srow * np.int32(n_b) + g * np.int32(_BR) + bcol)
                 * np.int32(v_size) + act)
        ew = _bits_to_exp(_threefry_bits(flatw.astype(jnp.uint32)))
        act_ref[0] = act
        sd_ref[0] = (ew / mvm) / ssum[None, :]


@functools.partial(jax.jit, static_argnums=())
def _run(probs, mask):
    n_b, v_size = probs.shape
    nt = -(-v_size // _TV)
    ng = n_b // _BR

    act_t, sd_t = pl.pallas_call(
        functools.partial(_body, nt, n_b, v_size),
        grid=(ng, nt),
        in_specs=[
            pl.BlockSpec((_BR, _TV), lambda g, t: (g, t)),
            pl.BlockSpec((_BR, _TV), lambda g, t: (g, t)),
        ],
        out_specs=[
            pl.BlockSpec((1, _S, _BR), lambda g, t: (g, 0, 0)),
            pl.BlockSpec((1, _S, _BR), lambda g, t: (g, 0, 0)),
        ],
        out_shape=[
            jax.ShapeDtypeStruct((ng, _S, _BR), jnp.int32),
            jax.ShapeDtypeStruct((ng, _S, _BR), jnp.float32),
        ],
        scratch_shapes=[
            pltpu.VMEM((_S, _BR, _TV), jnp.float32),
            pltpu.VMEM((_S, _BR, _TV), jnp.int32),
            pltpu.VMEM((_BR, _TV), jnp.float32),
        ],
        compiler_params=pltpu.CompilerParams(
            dimension_semantics=("arbitrary", "arbitrary")),
    )(probs, mask)
    sd = jnp.transpose(sd_t, (0, 2, 1)).reshape(n_b, _S)
    act = jnp.transpose(act_t, (0, 2, 1)).reshape(n_b, _S)
    return sd, act


def kernel(probs, mask, amt_samples):
    del amt_samples
    sd, act = _run(probs, mask)
    return (sd, act)


# speedup vs baseline: 1.0061x; 1.0061x over previous
import functools

import numpy as np
import jax
import jax.numpy as jnp
from jax.experimental import pallas as pl
from jax.experimental.pallas import tpu as pltpu

_S = 8
_BR = 8
_TV = 5120

_KS0 = np.uint32(0)
_KS1 = np.uint32(42)
_KS2 = np.uint32(int(_KS0) ^ int(_KS1) ^ 0x1BD11BDA)
_KSCHED = (_KS1, _KS2, _KS0, _KS1, _KS2, _KS0)
_ROT = ((13, 15, 26, 6), (17, 29, 16, 24))
_TINY = np.float32(np.finfo(np.float32).tiny)


def _threefry_bits(x1):
    x0 = jnp.zeros_like(x1)
    x1 = x1 + _KS1
    for i in range(5):
        for r in _ROT[i % 2]:
            x0 = x0 + x1
            x1 = ((x1 << np.uint32(r)) | (x1 >> np.uint32(32 - r))) ^ x0
        x0 = x0 + _KSCHED[i]
        x1 = x1 + np.uint32(int(_KSCHED[i + 1]) + i + 1)
    return x0 ^ x1


def _bits_to_exp(bits):
    fb = (bits >> np.uint32(9)) | np.uint32(0x3F800000)
    f = jax.lax.bitcast_convert_type(fb, jnp.float32) - np.float32(1.0)
    return -jnp.log(f + _TINY)


def _body(nt, n_b, v_size, p_ref, m_ref, act_ref, sd_ref,
          vmin_ref, vt_ref, rsum_ref):
    g = pl.program_id(0)
    t = pl.program_id(1)

    @pl.when(t == 0)
    def _init():
        vmin_ref[...] = jnp.full((_S, _BR, _TV), jnp.inf, jnp.float32)
        vt_ref[...] = jnp.zeros((_S, _BR, _TV), jnp.int32)
        rsum_ref[...] = jnp.zeros((_BR, _TV), jnp.float32)

    p = p_ref[...]
    m = m_ref[...]
    brow = jax.lax.broadcasted_iota(jnp.int32, (_BR, _TV), 0)
    lane = jax.lax.broadcasted_iota(jnp.int32, (_BR, _TV), 1)
    v = lane + t * np.int32(_TV)
    valid = v < np.int32(v_size)
    mf = jnp.where(v == 0, np.float32(1.0), m.astype(jnp.float32))
    pm = jnp.where(valid, (p + np.float32(1e-14)) * mf, np.float32(0.0))
    rsum_ref[...] += pm
    rpm = np.float32(1.0) / pm
    rowbase = (g * np.int32(_BR) + brow) * np.int32(v_size) + v
    for s in range(_S):
        flat = rowbase + np.int32(s * n_b * v_size)
        e = _bits_to_exp(_threefry_bits(flat.astype(jnp.uint32)))
        score = e * rpm
        old = vmin_ref[s]
        upd = score < old
        vt_ref[s] = jnp.where(upd, t, vt_ref[s])
        vmin_ref[s] = jnp.minimum(score, old)

    @pl.when(t == nt - 1)
    def _fin():
        ssum = jnp.sum(rsum_ref[...], axis=1)
        acts = []
        mvs = []
        for s in range(_S):
            vm = vmin_ref[s]
            mv = jnp.min(vm, axis=1)
            sel = vm == mv[:, None]
            cand = jnp.where(sel, vt_ref[s] * np.int32(_TV) + lane,
                             np.int32(2**31 - 1))
            acts.append(jnp.min(cand, axis=1))
            mvs.append(mv)
        act = jnp.stack(acts, axis=0)
        mvm = jnp.stack(mvs, axis=0)
        srow = jax.lax.broadcasted_iota(jnp.int32, (_S, _BR), 0)
        bcol = jax.lax.broadcasted_iota(jnp.int32, (_S, _BR), 1)
        flatw = ((srow * np.int32(n_b) + g * np.int32(_BR) + bcol)
                 * np.int32(v_size) + act)
        ew = _bits_to_exp(_threefry_bits(flatw.astype(jnp.uint32)))
        act_ref[0] = act
        sd_ref[0] = (ew / mvm) / ssum[None, :]


@functools.partial(jax.jit, static_argnums=())
def _run(probs, mask):
    n_b, v_size = probs.shape
    nt = -(-v_size // _TV)
    ng = n_b // _BR

    act_t, sd_t = pl.pallas_call(
        functools.partial(_body, nt, n_b, v_size),
        grid=(ng, nt),
        in_specs=[
            pl.BlockSpec((_BR, _TV), lambda g, t: (g, t)),
            pl.BlockSpec((_BR, _TV), lambda g, t: (g, t)),
        ],
        out_specs=[
            pl.BlockSpec((1, _S, _BR), lambda g, t: (g, 0, 0)),
            pl.BlockSpec((1, _S, _BR), lambda g, t: (g, 0, 0)),
        ],
        out_shape=[
            jax.ShapeDtypeStruct((ng, _S, _BR), jnp.int32),
            jax.ShapeDtypeStruct((ng, _S, _BR), jnp.float32),
        ],
        scratch_shapes=[
            pltpu.VMEM((_S, _BR, _TV), jnp.float32),
            pltpu.VMEM((_S, _BR, _TV), jnp.int32),
            pltpu.VMEM((_BR, _TV), jnp.float32),
        ],
        compiler_params=pltpu.CompilerParams(
            dimension_semantics=("arbitrary", "arbitrary")),
    )(probs, mask)
    sd = jnp.transpose(sd_t, (0, 2, 1)).reshape(n_b, _S)
    act = jnp.transpose(act_t, (0, 2, 1)).reshape(n_b, _S)
    return sd, act


def kernel(probs, mask, amt_samples):
    del amt_samples
    sd, act = _run(probs, mask)
    return (sd, act)


# speedup vs baseline: 1.0093x; 1.0032x over previous
import functools

import numpy as np
import jax
import jax.numpy as jnp
from jax.experimental import pallas as pl
from jax.experimental.pallas import tpu as pltpu

_S = 8
_BR = 8
_TV = 6144

_KS0 = np.uint32(0)
_KS1 = np.uint32(42)
_KS2 = np.uint32(int(_KS0) ^ int(_KS1) ^ 0x1BD11BDA)
_KSCHED = (_KS1, _KS2, _KS0, _KS1, _KS2, _KS0)
_ROT = ((13, 15, 26, 6), (17, 29, 16, 24))
_TINY = np.float32(np.finfo(np.float32).tiny)


def _threefry_bits(x1):
    x0 = jnp.zeros_like(x1)
    x1 = x1 + _KS1
    for i in range(5):
        for r in _ROT[i % 2]:
            x0 = x0 + x1
            x1 = ((x1 << np.uint32(r)) | (x1 >> np.uint32(32 - r))) ^ x0
        x0 = x0 + _KSCHED[i]
        x1 = x1 + np.uint32(int(_KSCHED[i + 1]) + i + 1)
    return x0 ^ x1


def _bits_to_exp(bits):
    fb = (bits >> np.uint32(9)) | np.uint32(0x3F800000)
    f = jax.lax.bitcast_convert_type(fb, jnp.float32) - np.float32(1.0)
    return -jnp.log(f + _TINY)


def _body(nt, n_b, v_size, p_ref, m_ref, act_ref, sd_ref,
          vmin_ref, vt_ref, rsum_ref):
    g = pl.program_id(0)
    t = pl.program_id(1)

    @pl.when(t == 0)
    def _init():
        vmin_ref[...] = jnp.full((_S, _BR, _TV), jnp.inf, jnp.float32)
        vt_ref[...] = jnp.zeros((_S, _BR, _TV), jnp.int32)
        rsum_ref[...] = jnp.zeros((_BR, _TV), jnp.float32)

    p = p_ref[...]
    m = m_ref[...]
    brow = jax.lax.broadcasted_iota(jnp.int32, (_BR, _TV), 0)
    lane = jax.lax.broadcasted_iota(jnp.int32, (_BR, _TV), 1)
    v = lane + t * np.int32(_TV)
    valid = v < np.int32(v_size)
    mf = jnp.where(v == 0, np.float32(1.0), m.astype(jnp.float32))
    pm = jnp.where(valid, (p + np.float32(1e-14)) * mf, np.float32(0.0))
    rsum_ref[...] += pm
    rpm = np.float32(1.0) / pm
    rowbase = (g * np.int32(_BR) + brow) * np.int32(v_size) + v
    for s in range(_S):
        flat = rowbase + np.int32(s * n_b * v_size)
        e = _bits_to_exp(_threefry_bits(flat.astype(jnp.uint32)))
        score = e * rpm
        old = vmin_ref[s]
        upd = score < old
        vt_ref[s] = jnp.where(upd, t, vt_ref[s])
        vmin_ref[s] = jnp.minimum(score, old)

    @pl.when(t == nt - 1)
    def _fin():
        ssum = jnp.sum(rsum_ref[...], axis=1)
        acts = []
        mvs = []
        for s in range(_S):
            vm = vmin_ref[s]
            mv = jnp.min(vm, axis=1)
            sel = vm == mv[:, None]
            cand = jnp.where(sel, vt_ref[s] * np.int32(_TV) + lane,
                             np.int32(2**31 - 1))
            acts.append(jnp.min(cand, axis=1))
            mvs.append(mv)
        act = jnp.stack(acts, axis=0)
        mvm = jnp.stack(mvs, axis=0)
        srow = jax.lax.broadcasted_iota(jnp.int32, (_S, _BR), 0)
        bcol = jax.lax.broadcasted_iota(jnp.int32, (_S, _BR), 1)
        flatw = ((srow * np.int32(n_b) + g * np.int32(_BR) + bcol)
                 * np.int32(v_size) + act)
        ew = _bits_to_exp(_threefry_bits(flatw.astype(jnp.uint32)))
        act_ref[0] = act
        sd_ref[0] = (ew / mvm) / ssum[None, :]


@functools.partial(jax.jit, static_argnums=())
def _run(probs, mask):
    n_b, v_size = probs.shape
    nt = -(-v_size // _TV)
    ng = n_b // _BR

    act_t, sd_t = pl.pallas_call(
        functools.partial(_body, nt, n_b, v_size),
        grid=(ng, nt),
        in_specs=[
            pl.BlockSpec((_BR, _TV), lambda g, t: (g, t)),
            pl.BlockSpec((_BR, _TV), lambda g, t: (g, t)),
        ],
        out_specs=[
            pl.BlockSpec((1, _S, _BR), lambda g, t: (g, 0, 0)),
            pl.BlockSpec((1, _S, _BR), lambda g, t: (g, 0, 0)),
        ],
        out_shape=[
            jax.ShapeDtypeStruct((ng, _S, _BR), jnp.int32),
            jax.ShapeDtypeStruct((ng, _S, _BR), jnp.float32),
        ],
        scratch_shapes=[
            pltpu.VMEM((_S, _BR, _TV), jnp.float32),
            pltpu.VMEM((_S, _BR, _TV), jnp.int32),
            pltpu.VMEM((_BR, _TV), jnp.float32),
        ],
        compiler_params=pltpu.CompilerParams(
            dimension_semantics=("arbitrary", "arbitrary")),
    )(probs, mask)
    sd = jnp.transpose(sd_t, (0, 2, 1)).reshape(n_b, _S)
    act = jnp.transpose(act_t, (0, 2, 1)).reshape(n_b, _S)
    return sd, act


def kernel(probs, mask, amt_samples):
    del amt_samples
    sd, act = _run(probs, mask)
    return (sd, act)
